# baseline (speedup 1.0000x reference)
import functools

import jax
import jax.numpy as jnp
from jax import lax
from jax.experimental import pallas as pl
from jax.experimental.pallas import tpu as pltpu
from jax.experimental.pallas import tpu_sc as plsc

_K = 16
_SHIFT = _K - 1
_NC, _NS = 2, 16
_CHUNK = 32
_NBUF = 3


def kernel(x):
    B, L, D = x.shape
    nw = _NC * _NS
    rows_per_w = (B * L) // nw
    workers_per_b = L // rows_per_w
    nchunks = rows_per_w // _CHUNK

    x2 = x.reshape(B * L, D)
    mesh = plsc.VectorSubcoreMesh(core_axis_name="c", subcore_axis_name="s")

    @functools.partial(
        pl.kernel,
        mesh=mesh,
        out_type=jax.ShapeDtypeStruct((B, L, 2, D), x.dtype),
        scratch_types=(
            [pltpu.VMEM((_CHUNK, D), x.dtype) for _ in range(_NBUF)]
            + [pltpu.VMEM((_CHUNK,), jnp.int32), pltpu.VMEM((_SHIFT, D), x.dtype)]
            + [pltpu.SemaphoreType.DMA for _ in range(2 * _NBUF)]
        ),
    )
    def span_sc(x_hbm, out_hbm, b0, b1, b2, idx_v, zbuf, i0, i1, i2, o0, o1, o2):
        wid = lax.axis_index("s") * _NC + lax.axis_index("c")
        b = wid // workers_per_b
        s = (wid % workers_per_b) * rows_per_w
        base = b * L + s
        bufs = (b0, b1, b2)
        ins = (i0, i1, i2)
        outs = (o0, o1, o2)

        def in_copy(g):
            return pltpu.make_async_copy(
                x_hbm.at[pl.ds(base + g * _CHUNK, _CHUNK)],
                bufs[g % _NBUF],
                ins[g % _NBUF],
            )

        def out0_copy(g):
            return pltpu.make_async_copy(
                bufs[g % _NBUF],
                out_hbm.at[b, pl.ds(s + g * _CHUNK, _CHUNK), 0],
                outs[g % _NBUF],
            )

        def out1_copy(g):
            return pltpu.make_async_copy(
                bufs[g % _NBUF],
                out_hbm.at[b, pl.ds(s + g * _CHUNK - _SHIFT, _CHUNK), 1],
                outs[g % _NBUF],
            )

        for g in range(_NBUF):
            in_copy(g).start()
        for g in range(nchunks):
            in_copy(g).wait()
            out0_copy(g).start()
            o1_ok = jnp.logical_or(g > 0, s > 0)

            @pl.when(o1_ok)
            def _():
                out1_copy(g).start()

            out0_copy(g).wait()

            @pl.when(o1_ok)
            def _():
                out1_copy(g).wait()

            if g + _NBUF < nchunks:
                in_copy(g + _NBUF).start()

        @pl.when(s == 0)
        def _():
            for j in range(_CHUNK // 16):
                idx_v[pl.ds(j * 16, 16)] = (
                    lax.iota(jnp.int32, 16) + (b * L + _SHIFT + j * 16)
                )
            pltpu.async_copy(x_hbm.at[idx_v], b0, i0).wait()
            pltpu.sync_copy(b0, out_hbm.at[b, pl.ds(0, _CHUNK), 1])

        @pl.when(wid % workers_per_b == workers_per_b - 1)
        def _():
            zero = jnp.zeros((16,), x.dtype)

            def zrow(i, carry):
                zbuf[i // (D // 16), pl.ds((i % (D // 16)) * 16, 16)] = zero
                return carry

            lax.fori_loop(0, (_SHIFT * D) // 16, zrow, 0)
            pltpu.sync_copy(zbuf, out_hbm.at[b, pl.ds(L - _SHIFT, _SHIFT), 1])

    return span_sc(x2)


# speedup vs baseline: 1.0175x; 1.0175x over previous
import functools

import jax
import jax.numpy as jnp
from jax import lax
from jax.experimental import pallas as pl
from jax.experimental.pallas import tpu as pltpu
from jax.experimental.pallas import tpu_sc as plsc

_K = 16
_SHIFT = _K - 1
_NC, _NS = 2, 16
_CHUNK = 64


def kernel(x):
    B, L, D = x.shape
    nw = _NC * _NS
    rows_per_w = (B * L) // nw
    workers_per_b = L // rows_per_w
    nchunks = rows_per_w // _CHUNK

    x2 = x.reshape(B * L, D)
    mesh = plsc.VectorSubcoreMesh(core_axis_name="c", subcore_axis_name="s")

    @functools.partial(
        pl.kernel,
        mesh=mesh,
        out_type=jax.ShapeDtypeStruct((B, L, 2, D), x.dtype),
        scratch_types=[
            pltpu.VMEM((_CHUNK, D), x.dtype),
            pltpu.VMEM((_CHUNK,), jnp.int32),
            pltpu.VMEM((_SHIFT, D), x.dtype),
            pltpu.SemaphoreType.DMA,
        ],
    )
    def span_sc(x_hbm, out_hbm, buf, idx_v, zbuf, sem):
        wid = lax.axis_index("s") * _NC + lax.axis_index("c")
        b = wid // workers_per_b
        s = (wid % workers_per_b) * rows_per_w

        def chunk_body(g, carry):
            r = s + g * _CHUNK
            pltpu.sync_copy(x_hbm.at[pl.ds(b * L + r, _CHUNK)], buf)
            pltpu.sync_copy(buf, out_hbm.at[b, pl.ds(r, _CHUNK), 0])

            first = r == 0

            @pl.when(first)
            def _():
                for j in range(_CHUNK // 16):
                    idx_v[pl.ds(j * 16, 16)] = (
                        lax.iota(jnp.int32, 16) + (b * L + _SHIFT + j * 16)
                    )
                pltpu.async_copy(x_hbm.at[idx_v], buf, sem).wait()
                pltpu.sync_copy(buf, out_hbm.at[b, pl.ds(0, _CHUNK), 1])

            @pl.when(jnp.logical_not(first))
            def _():
                pltpu.sync_copy(
                    buf, out_hbm.at[b, pl.ds(r - _SHIFT, _CHUNK), 1]
                )

            return carry

        lax.fori_loop(0, nchunks, chunk_body, 0)

        @pl.when(wid % workers_per_b == workers_per_b - 1)
        def _():
            zero = jnp.zeros((16,), x.dtype)

            def zrow(i, carry):
                zbuf[i // (D // 16), pl.ds((i % (D // 16)) * 16, 16)] = zero
                return carry

            lax.fori_loop(0, (_SHIFT * D) // 16, zrow, 0)
            pltpu.sync_copy(zbuf, out_hbm.at[b, pl.ds(L - _SHIFT, _SHIFT), 1])

    return span_sc(x2)
